# Initial kernel scaffold; baseline (speedup 1.0000x reference)
import jax
import jax.numpy as jnp
from jax.experimental import pallas as pl


def kernel(x, emb, pos_emb):
    raise NotImplementedError("write your pallas kernel here")


# speedup vs baseline: 1.4951x; 1.4951x over previous
import functools

import jax
import jax.numpy as jnp
from jax import lax
from jax.experimental import pallas as pl
from jax.experimental.pallas import tpu as pltpu
from jax.experimental.pallas import tpu_sc as plsc

NC = 2
NS = 16
NW = NC * NS
LANES = 16


def _make_kernel(N, V, H, S, C, NBUF):
    per_w = N // NW
    n_chunks = per_w // C
    assert per_w % C == 0 and n_chunks % NBUF == 0
    LAG = 2

    mesh = plsc.VectorSubcoreMesh(core_axis_name="c", subcore_axis_name="s")

    @functools.partial(
        pl.kernel,
        out_type=jax.ShapeDtypeStruct((N, H), jnp.float32),
        mesh=mesh,
        compiler_params=pltpu.CompilerParams(use_tc_tiling_on_sc=False),
        scratch_types=dict(
            idx_v=pltpu.VMEM((per_w,), jnp.int32),
            pos_v=pltpu.VMEM((S, H), jnp.float32),
            rows_v=pltpu.VMEM((NBUF, C, H), jnp.float32),
            gsems=[pltpu.SemaphoreType.DMA] * NBUF,
            osems=[pltpu.SemaphoreType.DMA] * NBUF,
        ),
    )
    def body(idx_hbm, emb_hbm, pos_hbm, out_hbm, idx_v, pos_v, rows_v,
             gsems, osems):
        wid = lax.axis_index("s") * NC + lax.axis_index("c")
        base = wid * per_w

        pltpu.sync_copy(idx_hbm.at[pl.ds(base, per_w)], idx_v)
        pltpu.sync_copy(pos_hbm, pos_v)

        def start_gather(chunk, b):
            pltpu.async_copy(
                emb_hbm.at[idx_v.at[pl.ds(chunk * C, C)]],
                rows_v.at[b], gsems[b])

        for b in range(NBUF - LAG):
            start_gather(b, b)

        @pl.loop(0, n_chunks, step=NBUF)
        def group(c0):
            for b in range(NBUF):
                c = c0 + b
                bp = (b - LAG) % NBUF
                cp = c + NBUF - LAG

                @pl.when(cp < n_chunks)
                def _():
                    @pl.when(c >= LAG)
                    def _():
                        pltpu.make_async_copy(
                            rows_v.at[bp],
                            out_hbm.at[pl.ds(base + (c - LAG) * C, C)],
                            osems[bp]).wait()
                    start_gather(cp, bp)

                pltpu.make_async_copy(
                    emb_hbm.at[idx_v.at[pl.ds(c * C, C)]],
                    rows_v.at[b], gsems[b]).wait()

                @pl.loop(0, C, unroll=8)
                def add_pos(r):
                    plsc.addupdate(rows_v.at[b, r, pl.ds(0, LANES)],
                                   pos_v[r, pl.ds(0, LANES)])
                    plsc.addupdate(rows_v.at[b, r, pl.ds(LANES, LANES)],
                                   pos_v[r, pl.ds(LANES, LANES)])

                pltpu.async_copy(
                    rows_v.at[b],
                    out_hbm.at[pl.ds(base + c * C, C)], osems[b])

        for b in range(NBUF):
            c_last = n_chunks - NBUF + b
            pltpu.make_async_copy(
                rows_v.at[b],
                out_hbm.at[pl.ds(base + c_last * C, C)], osems[b]).wait()

    return body


def kernel(x, emb, pos_emb):
    B, S = x.shape
    V, H = emb.shape
    N = B * S
    idx = x.reshape(N)
    fn = _make_kernel(N, V, H, S, C=S, NBUF=8)
    out = fn(idx, emb, pos_emb)
    return out.reshape(B, S, H)
